# baseline (speedup 1.0000x reference)
import jax
import jax.numpy as jnp
from jax import lax
from jax.experimental import pallas as pl
from jax.experimental.pallas import tpu as pltpu

E = 64
RANK = 16
D_MODEL = 1024
D_FF = 2048
ALPHA = 2.0
S = 2048

ROW_TILE = 128


def _dot_t(a, b):
    return lax.dot_general(a, b, (((1,), (1,)), ((), ())),
                           preferred_element_type=jnp.float32)


def _dot(a, b):
    return lax.dot_general(a, b, (((1,), (0,)), ((), ())),
                           preferred_element_type=jnp.float32)


def _router_body(x_ref, wr_ref, sel_ref, coef_ref):
    x = x_ref[...]
    logits = _dot_t(x, wr_ref[...])
    m = jnp.max(logits, axis=1, keepdims=True)
    eids = lax.broadcasted_iota(jnp.int32, logits.shape, 1)
    sel_ref[...] = jnp.min(jnp.where(logits >= m, eids, E), axis=1,
                           keepdims=True)
    p = jnp.exp(m - jnp.max(m))
    coef_ref[...] = p / jnp.sum(p)


def _upgate_body(x_ref, sel_ref, wu_ref, wg_ref, au_ref, ag_ref,
                 bu_ref, bg_ref, hid_ref, wu_s, wg_s, au_s, ag_s):
    t = pl.program_id(0)

    @pl.when(t == 0)
    def _():
        wu_s[...] = wu_ref[...].astype(jnp.bfloat16)
        wg_s[...] = wg_ref[...].astype(jnp.bfloat16)
        au_s[...] = au_ref[...].astype(jnp.bfloat16)
        ag_s[...] = ag_ref[...].astype(jnp.bfloat16)

    xb = x_ref[...].astype(jnp.bfloat16)
    sel = sel_ref[...]
    h1 = _dot_t(xb, wu_s[...])
    h3 = _dot_t(xb, wg_s[...])
    pu = _dot_t(xb, au_s[...])
    pg = _dot_t(xb, ag_s[...])
    mask = (lax.broadcasted_iota(jnp.int32, pu.shape, 1) // RANK) == sel
    l1 = _dot(jnp.where(mask, pu, 0.0).astype(jnp.bfloat16), bu_ref[...])
    l3 = _dot(jnp.where(mask, pg, 0.0).astype(jnp.bfloat16), bg_ref[...])
    a = h1 + ALPHA * l1
    b = h3 + ALPHA * l3
    hid_ref[...] = (a * jax.nn.sigmoid(a) * b).astype(jnp.bfloat16)


def _down_body(hid_ref, sel_ref, coef_ref, wd_ref, ad_ref, bd_ref,
               out_ref, wd_s, ad_s):
    t = pl.program_id(0)

    @pl.when(t == 0)
    def _():
        wd_s[...] = wd_ref[...].astype(jnp.bfloat16)
        ad_s[...] = ad_ref[...].astype(jnp.bfloat16)

    hb = hid_ref[...]
    sel = sel_ref[...]
    coef = coef_ref[...]
    qd = _dot_t(hb, ad_s[...])
    mask = (lax.broadcasted_iota(jnp.int32, qd.shape, 1) // RANK) == sel
    l2 = _dot(jnp.where(mask, qd, 0.0).astype(jnp.bfloat16), bd_ref[...])
    out_ref[...] = coef * (_dot_t(hb, wd_s[...]) + ALPHA * l2)


@jax.jit
def _run(x, W_up, W_gate_proj, W_down, W_router,
         up_A, up_B, down_A, down_B, gate_A, gate_B):
    sel, coef = pl.pallas_call(
        _router_body,
        out_shape=(jax.ShapeDtypeStruct((S, 1), jnp.int32),
                   jax.ShapeDtypeStruct((S, 1), jnp.float32)),
    )(x, W_router)

    bf = jnp.bfloat16
    au = up_A.reshape(E * RANK, D_MODEL)
    ag = gate_A.reshape(E * RANK, D_MODEL)
    ad = down_A.reshape(E * RANK, D_FF)
    bu = up_B.transpose(0, 2, 1).reshape(E * RANK, D_FF).astype(bf)
    bg = gate_B.transpose(0, 2, 1).reshape(E * RANK, D_FF).astype(bf)
    bd = down_B.transpose(0, 2, 1).reshape(E * RANK, D_MODEL).astype(bf)

    n_tiles = S // ROW_TILE
    row = lambda t: (t, 0)
    full = lambda t: (0, 0)
    arb = pltpu.CompilerParams(dimension_semantics=("arbitrary",))

    hidden = pl.pallas_call(
        _upgate_body,
        grid=(n_tiles,),
        in_specs=[
            pl.BlockSpec((ROW_TILE, D_MODEL), row),
            pl.BlockSpec((ROW_TILE, 1), row),
            pl.BlockSpec((D_FF, D_MODEL), full),
            pl.BlockSpec((D_FF, D_MODEL), full),
            pl.BlockSpec((E * RANK, D_MODEL), full),
            pl.BlockSpec((E * RANK, D_MODEL), full),
            pl.BlockSpec((E * RANK, D_FF), full),
            pl.BlockSpec((E * RANK, D_FF), full),
        ],
        out_specs=pl.BlockSpec((ROW_TILE, D_FF), row),
        out_shape=jax.ShapeDtypeStruct((S, D_FF), jnp.bfloat16),
        scratch_shapes=[
            pltpu.VMEM((D_FF, D_MODEL), bf),
            pltpu.VMEM((D_FF, D_MODEL), bf),
            pltpu.VMEM((E * RANK, D_MODEL), bf),
            pltpu.VMEM((E * RANK, D_MODEL), bf),
        ],
        compiler_params=arb,
    )(x, sel, W_up, W_gate_proj, au, ag, bu, bg)

    out = pl.pallas_call(
        _down_body,
        grid=(n_tiles,),
        in_specs=[
            pl.BlockSpec((ROW_TILE, D_FF), row),
            pl.BlockSpec((ROW_TILE, 1), row),
            pl.BlockSpec((ROW_TILE, 1), row),
            pl.BlockSpec((D_MODEL, D_FF), full),
            pl.BlockSpec((E * RANK, D_FF), full),
            pl.BlockSpec((E * RANK, D_MODEL), full),
        ],
        out_specs=pl.BlockSpec((ROW_TILE, D_MODEL), row),
        out_shape=jax.ShapeDtypeStruct((S, D_MODEL), jnp.float32),
        scratch_shapes=[
            pltpu.VMEM((D_MODEL, D_FF), bf),
            pltpu.VMEM((E * RANK, D_FF), bf),
        ],
        compiler_params=arb,
    )(hidden, sel, coef, W_down, ad, bd)
    return out


def kernel(inputs, W_up, W_gate_proj, W_down, W_router,
           up_A, up_B, down_A, down_B, gate_A, gate_B):
    x = inputs.reshape(S, D_MODEL)
    out = _run(x, W_up, W_gate_proj, W_down, W_router,
               up_A, up_B, down_A, down_B, gate_A, gate_B)
    return out.reshape(1, S, D_MODEL)


# speedup vs baseline: 1.7402x; 1.7402x over previous
import jax
import jax.numpy as jnp
from jax import lax
from jax.experimental import pallas as pl
from jax.experimental.pallas import tpu as pltpu

E = 64
RANK = 16
D_MODEL = 1024
D_FF = 2048
ALPHA = 2.0
S = 2048

FC = 256
N_FC = D_FF // FC
DC = 256
BF = jnp.bfloat16


def _dot_t(a, b):
    return lax.dot_general(a, b, (((1,), (1,)), ((), ())),
                           preferred_element_type=jnp.float32)


def _dot(a, b):
    return lax.dot_general(a, b, (((1,), (0,)), ((), ())),
                           preferred_element_type=jnp.float32)


def _router_body(x_ref, wr_ref, sel_ref, coef_ref, xb_ref):
    x = x_ref[...]
    logits = _dot_t(x, wr_ref[...])
    m = jnp.max(logits, axis=1, keepdims=True)
    eids = lax.broadcasted_iota(jnp.int32, logits.shape, 1)
    sel_ref[...] = jnp.min(jnp.where(logits >= m, eids, E), axis=1,
                           keepdims=True)
    p = jnp.exp(m - jnp.max(m))
    coef_ref[...] = p / jnp.sum(p)
    xb_ref[...] = x.astype(BF)


def _upgate_body(xb_ref, sel_ref, au_ref, ag_ref, wu_ref, wg_ref,
                 bu_ref, bg_ref, hid_ref, pmu_s, pmg_s):
    c = pl.program_id(0)

    @pl.when(c == 0)
    def _():
        xb = xb_ref[...]
        sel = sel_ref[...]
        pu = _dot_t(xb, au_ref[...].astype(BF))
        pg = _dot_t(xb, ag_ref[...].astype(BF))
        mask = (lax.broadcasted_iota(jnp.int32, pu.shape, 1) // RANK) == sel
        pmu_s[...] = jnp.where(mask, pu, 0.0).astype(BF)
        pmg_s[...] = jnp.where(mask, pg, 0.0).astype(BF)

    xb = xb_ref[...]
    h1 = _dot_t(xb, wu_ref[...].astype(BF))
    h3 = _dot_t(xb, wg_ref[...].astype(BF))
    l1 = _dot(pmu_s[...], bu_ref[...])
    l3 = _dot(pmg_s[...], bg_ref[...])
    a = h1 + ALPHA * l1
    b = h3 + ALPHA * l3
    hid_ref[...] = (a * jax.nn.sigmoid(a) * b).astype(BF)


def _down_body(hid_ref, sel_ref, coef_ref, ad_ref, wd_ref, bd_ref,
               out_ref, wacc_s, qacc_s):
    c = pl.program_id(0)

    @pl.when(c == 0)
    def _():
        wacc_s[...] = jnp.zeros_like(wacc_s)
        qacc_s[...] = jnp.zeros_like(qacc_s)

    @pl.when(c < N_FC)
    def _():
        hb = hid_ref[...]
        wacc_s[...] += _dot_t(hb, wd_ref[...].astype(BF))
        qacc_s[...] += _dot_t(hb, ad_ref[...].astype(BF))

    @pl.when(c == N_FC)
    def _():
        sel = sel_ref[...]
        qd = qacc_s[...]
        mask = (lax.broadcasted_iota(jnp.int32, qd.shape, 1) // RANK) == sel
        qm = jnp.where(mask, qd, 0.0).astype(BF)
        l2 = _dot(qm, bd_ref[...])
        out_ref[...] = coef_ref[...] * (wacc_s[...] + ALPHA * l2)


@jax.jit
def _run(x, W_up, W_gate_proj, W_down, W_router,
         up_A, up_B, down_A, down_B, gate_A, gate_B):
    sel, coef, xb = pl.pallas_call(
        _router_body,
        out_shape=(jax.ShapeDtypeStruct((S, 1), jnp.int32),
                   jax.ShapeDtypeStruct((S, 1), jnp.float32),
                   jax.ShapeDtypeStruct((S, D_MODEL), BF)),
    )(x, W_router)

    au = up_A.reshape(E * RANK, D_MODEL)
    ag = gate_A.reshape(E * RANK, D_MODEL)
    ad = down_A.reshape(E * RANK, D_FF)
    bu = up_B.transpose(0, 2, 1).reshape(E * RANK, D_FF).astype(BF)
    bg = gate_B.transpose(0, 2, 1).reshape(E * RANK, D_FF).astype(BF)
    bd = down_B.transpose(0, 2, 1).reshape(E * RANK, D_MODEL).astype(BF)

    full = lambda c: (0, 0)
    arb = pltpu.CompilerParams(dimension_semantics=("arbitrary",))

    hidden = pl.pallas_call(
        _upgate_body,
        grid=(N_FC,),
        in_specs=[
            pl.BlockSpec((S, D_MODEL), full),
            pl.BlockSpec((S, 1), full),
            pl.BlockSpec((E * RANK, D_MODEL), full),
            pl.BlockSpec((E * RANK, D_MODEL), full),
            pl.BlockSpec((FC, D_MODEL), lambda c: (c, 0)),
            pl.BlockSpec((FC, D_MODEL), lambda c: (c, 0)),
            pl.BlockSpec((E * RANK, FC), lambda c: (0, c)),
            pl.BlockSpec((E * RANK, FC), lambda c: (0, c)),
        ],
        out_specs=pl.BlockSpec((S, FC), lambda c: (0, c)),
        out_shape=jax.ShapeDtypeStruct((S, D_FF), BF),
        scratch_shapes=[
            pltpu.VMEM((S, E * RANK), BF),
            pltpu.VMEM((S, E * RANK), BF),
        ],
        compiler_params=arb,
    )(xb, sel, au, ag, W_up, W_gate_proj, bu, bg)

    cl = lambda c: (jnp.minimum(c, N_FC - 1),)
    out = pl.pallas_call(
        _down_body,
        grid=(N_FC + 1,),
        in_specs=[
            pl.BlockSpec((S, FC), lambda c: (0, jnp.minimum(c, N_FC - 1))),
            pl.BlockSpec((S, 1), full),
            pl.BlockSpec((S, 1), full),
            pl.BlockSpec((E * RANK, FC),
                         lambda c: (0, jnp.minimum(c, N_FC - 1))),
            pl.BlockSpec((D_MODEL, FC),
                         lambda c: (0, jnp.minimum(c, N_FC - 1))),
            pl.BlockSpec((E * RANK, D_MODEL), full),
        ],
        out_specs=pl.BlockSpec((S, D_MODEL), full),
        out_shape=jax.ShapeDtypeStruct((S, D_MODEL), jnp.float32),
        scratch_shapes=[
            pltpu.VMEM((S, D_MODEL), jnp.float32),
            pltpu.VMEM((S, E * RANK), jnp.float32),
        ],
        compiler_params=arb,
    )(hidden, sel, coef, ad, W_down, bd)
    return out


def kernel(inputs, W_up, W_gate_proj, W_down, W_router,
           up_A, up_B, down_A, down_B, gate_A, gate_B):
    x = inputs.reshape(S, D_MODEL)
    out = _run(x, W_up, W_gate_proj, W_down, W_router,
               up_A, up_B, down_A, down_B, gate_A, gate_B)
    return out.reshape(1, S, D_MODEL)
